# baseline (speedup 1.0000x reference)
import functools
import math

import jax
import jax.numpy as jnp
from jax import lax
from jax.experimental import pallas as pl
from jax.experimental.pallas import tpu as pltpu
from jax.experimental.pallas import tpu_sc as plsc

B, N, D = 64, 256, 256
L, H = 2, 2
DH = D // H
FF = 512
E = 262144
BN = B * N

NC, NS = 2, 16
NW = NC * NS
CC = 80
NCH = 7
GC = NCH * CC
EW = E // NW
KB = 128
NB = EW // KB
ZR = 128
RPT = BN // NS


def _leaky(x):
    return jnp.where(x >= 0, x, 0.01 * x)


def _ln(x, g, b):
    m = jnp.mean(x, axis=-1, keepdims=True)
    v = jnp.mean((x - m) ** 2, axis=-1, keepdims=True)
    return (x - m) / jnp.sqrt(v + 1e-5) * g + b


def _transformer_body(wch_ref, x_ref, wq_ref, wk_ref, wv_ref, wo_ref,
                      w1_ref, w2_ref, l1g_ref, l1b_ref, l2g_ref, l2b_ref,
                      zch_ref):
    def mm(a, b):
        return lax.dot_general(
            a.astype(jnp.bfloat16), b.astype(jnp.bfloat16),
            (((1,), (0,)), ((), ())), preferred_element_type=jnp.float32)

    z = x_ref[0]
    impacc = jnp.zeros((1, N), jnp.float32)
    scale = 1.0 / math.sqrt(DH)
    for l in range(L):
        q = mm(z, wq_ref[l])
        k = mm(z, wk_ref[l])
        v = mm(z, wv_ref[l])
        heads = []
        for h in range(H):
            sl = slice(h * DH, (h + 1) * DH)
            qh, kh, vh = q[:, sl], k[:, sl], v[:, sl]
            s = lax.dot_general(
                qh.astype(jnp.bfloat16), kh.astype(jnp.bfloat16),
                (((1,), (1,)), ((), ())),
                preferred_element_type=jnp.float32) * scale
            m = jnp.max(s, axis=-1, keepdims=True)
            e = jnp.exp(s - m)
            den = jnp.sum(e, axis=-1, keepdims=True)
            a = e / den
            impacc = impacc + wch_ref[l * H + h] * jnp.sum(a, axis=0,
                                                           keepdims=True)
            heads.append(mm(a, vh))
        o = mm(jnp.concatenate(heads, axis=1), wo_ref[l])
        z = _ln(z + o, l1g_ref[l], l1b_ref[l])
        f = mm(_leaky(mm(z, w1_ref[l])), w2_ref[l])
        z = _ln(z + f, l2g_ref[l], l2b_ref[l])
    imp = _leaky(impacc * (1.0 / N))
    impc = imp.reshape(N, 1)
    zfw = z * impc
    zcat = jnp.concatenate(
        [z, zfw, impc, jnp.ones((N, 1), jnp.float32),
         jnp.zeros((N, GC - 2 * D - 2), jnp.float32)], axis=1)
    for p in range(NCH):
        zch_ref[p] = zcat[:, p * CC:(p + 1) * CC]


def _run_transformer(x, wq, wk, wv, wo, w1, w2, l1g, l1b, l2g, l2b, w_ch):
    full = lambda a: pl.BlockSpec(a.shape, lambda b: (0,) * a.ndim)
    return pl.pallas_call(
        _transformer_body,
        grid=(B,),
        in_specs=[
            pl.BlockSpec(memory_space=pltpu.SMEM),
            pl.BlockSpec((1, N, D), lambda b: (b, 0, 0)),
            full(wq), full(wk), full(wv), full(wo), full(w1), full(w2),
            full(l1g), full(l1b), full(l2g), full(l2b),
        ],
        out_specs=pl.BlockSpec((NCH, N, CC), lambda b: (0, b, 0)),
        out_shape=jax.ShapeDtypeStruct((NCH, BN, CC), jnp.float32),
    )(w_ch, x, wq, wk, wv, wo, w1, w2, l1g, l1b, l2g, l2b)


_NGB = 1
_NJ = NB // (2 * _NGB)


def _sc_body(*refs):
    tabs = refs[:NCH]
    (gidx_hbm, sidx_hbm, out_hbm,
     gidx_v, sidx_v, rows_v, acc_s, sem) = refs[NCH:]
    cid = lax.axis_index("c")
    sid = lax.axis_index("s")
    wid = sid * NC + cid

    pltpu.sync_copy(gidx_hbm.at[wid], gidx_v)
    pltpu.sync_copy(sidx_hbm.at[wid], sidx_v)

    for p in range(NCH):
        @pl.loop(0, ZR)
        def _zf(i):
            for j in range(CC // 16):
                rows_v[0, i, pl.ds(j * 16, 16)] = jnp.zeros((16,), jnp.float32)

        for r in range(RPT // ZR):
            pltpu.sync_copy(rows_v.at[0], acc_s.at[pl.ds(sid * RPT + r * ZR, ZR)])
        plsc.subcore_barrier()

        def fire_g(bi, t):
            pltpu.async_copy(tabs[p].at[gidx_v.at[bi]], rows_v.at[t], sem)

        def drain_g(t):
            pltpu.make_async_copy(tabs[p].at[gidx_v.at[0]], rows_v.at[t],
                                  sem).wait()

        def scat(bi, t):
            pltpu.sync_copy(rows_v.at[t], acc_s.at[sidx_v.at[bi]], add=True)

        for t in range(_NGB):
            fire_g(t, t)

        @pl.loop(0, _NJ)
        def _eb(jj):
            b0 = 2 * _NGB * jj
            for t in range(_NGB):
                drain_g(t)
            for t in range(_NGB):
                fire_g(b0 + _NGB + t, _NGB + t)
            for t in range(_NGB):
                scat(b0 + t, t)
            for t in range(_NGB):
                drain_g(_NGB + t)

            @pl.when(jj + 1 < _NJ)
            def _():
                for t in range(_NGB):
                    fire_g(b0 + 2 * _NGB + t, t)

            for t in range(_NGB):
                scat(b0 + _NGB + t, _NGB + t)

        plsc.subcore_barrier()
        pltpu.sync_copy(acc_s.at[pl.ds(sid * RPT, RPT)],
                        out_hbm.at[cid, p, pl.ds(sid * RPT, RPT)])


def _run_sc_phase(tables, gidx, sidx):
    mesh = plsc.VectorSubcoreMesh(core_axis_name="c", subcore_axis_name="s",
                                  num_cores=NC, num_subcores=NS)
    f = pl.kernel(
        _sc_body,
        out_type=jax.ShapeDtypeStruct((NC, NCH, BN, CC), jnp.float32),
        mesh=mesh,
        scratch_types=[
            pltpu.VMEM((NB, KB), jnp.int32),
            pltpu.VMEM((NB, KB), jnp.int32),
            pltpu.VMEM((2 * _NGB, KB, CC), jnp.float32),
            pltpu.VMEM_SHARED((BN, CC), jnp.float32),
            pltpu.SemaphoreType.DMA,
        ],
        compiler_params=pltpu.CompilerParams(use_tc_tiling_on_sc=False),
    )
    return f(*tables, gidx.reshape(NW, NB, KB), sidx.reshape(NW, NB, KB))


_RB = 1024


def _phase_b_body(wch_ref, pa_ref, out_ref):
    pa = pa_ref[...]
    s = pa[0] + pa[1]
    cat = jnp.concatenate([s[p] for p in range(NCH)], axis=1)
    s1 = cat[:, :D]
    m2 = cat[:, D:2 * D]
    sw2 = cat[:, 2 * D:2 * D + 1]
    cnt = cat[:, 2 * D + 1:2 * D + 2]
    c = _leaky((wch_ref[0] + wch_ref[1] + wch_ref[2] + wch_ref[3]) / N)
    agg1 = (c * s1) / (c * cnt + 1e-6)
    agg2 = m2 / (sw2 + 1e-6)
    outcat = jnp.concatenate(
        [agg1, agg2, jnp.zeros((_RB, 1), jnp.float32),
         jnp.ones((_RB, 1), jnp.float32),
         jnp.zeros((_RB, GC - 2 * D - 2), jnp.float32)], axis=1)
    for p in range(NCH):
        out_ref[p] = outcat[:, p * CC:(p + 1) * CC]


def _run_phase_b(pa, w_ch):
    nr = BN // _RB
    return pl.pallas_call(
        _phase_b_body,
        grid=(nr,),
        in_specs=[
            pl.BlockSpec(memory_space=pltpu.SMEM),
            pl.BlockSpec((NC, NCH, _RB, CC), lambda r: (0, 0, r, 0)),
        ],
        out_specs=pl.BlockSpec((NCH, _RB, CC), lambda r: (0, r, 0)),
        out_shape=jax.ShapeDtypeStruct((NCH, BN, CC), jnp.float32),
    )(w_ch, pa)


def _phase_d_body(wch_ref, pc_ref, z4_ref, wh1_ref, wh2_ref, out_ref):
    pc = pc_ref[...]
    s = pc[0] + pc[1]
    cat = jnp.concatenate([s[p] for p in range(NCH)], axis=1)
    t1 = cat[:, :D]
    u2 = cat[:, D:2 * D]
    cnt = cat[:, 2 * D + 1:2 * D + 2]
    iloc = 2 * D - (NCH - 1) * CC
    im = z4_ref[...][:, iloc:iloc + 1]
    c = _leaky((wch_ref[0] + wch_ref[1] + wch_ref[2] + wch_ref[3]) / N)
    o1 = (c * t1) / (c * cnt + 1e-6)
    o2 = (im * u2) / (im * cnt + 1e-6)
    out_ref[...] = _leaky(o1 @ wh1_ref[...]) + _leaky(o2 @ wh2_ref[...])


def _run_phase_d(pc, z4, wh1, wh2, w_ch):
    nr = BN // _RB
    full = lambda a: pl.BlockSpec(a.shape, lambda r: (0,) * a.ndim)
    return pl.pallas_call(
        _phase_d_body,
        grid=(nr,),
        in_specs=[
            pl.BlockSpec(memory_space=pltpu.SMEM),
            pl.BlockSpec((NC, NCH, _RB, CC), lambda r: (0, 0, r, 0)),
            pl.BlockSpec((_RB, CC), lambda r: (r, 0)),
            full(wh1), full(wh2),
        ],
        out_specs=pl.BlockSpec((_RB, D), lambda r: (r, 0)),
        out_shape=jax.ShapeDtypeStruct((BN, D), jnp.float32),
    )(w_ch, pc, z4, wh1, wh2)


def kernel(x, edge_index, edge_weight, batch, Wq, Wk, Wv, Wo, W1, W2,
           ln1g, ln1b, ln2g, ln2b, w_ch, Wh1, Wh2):
    del edge_weight, batch
    src = edge_index[0]
    he = edge_index[1]
    zch = _run_transformer(x, Wq, Wk, Wv, Wo, W1, W2,
                           ln1g, ln1b, ln2g, ln2b, w_ch)
    ztabs = [zch[p] for p in range(NCH)]
    pa = _run_sc_phase(ztabs, src, he)
    aggc = _run_phase_b(pa, w_ch)
    atabs = [aggc[p] for p in range(NCH)]
    pc = _run_sc_phase(atabs, he, src)
    return _run_phase_d(pc, zch[NCH - 1], Wh1, Wh2, w_ch)


# speedup vs baseline: 1.1282x; 1.1282x over previous
import functools
import math

import jax
import jax.numpy as jnp
from jax import lax
from jax.experimental import pallas as pl
from jax.experimental.pallas import tpu as pltpu
from jax.experimental.pallas import tpu_sc as plsc

B, N, D = 64, 256, 256
L, H = 2, 2
DH = D // H
FF = 512
E = 262144
BN = B * N

NC, NS = 2, 16
NW = NC * NS
CC = 80
NCH = 7
GC = NCH * CC
EW = E // NW
KB = 128
NB = EW // KB
ZR = 128
RPT = BN // NS


def _leaky(x):
    return jnp.where(x >= 0, x, 0.01 * x)


def _ln(x, g, b):
    m = jnp.mean(x, axis=-1, keepdims=True)
    v = jnp.mean((x - m) ** 2, axis=-1, keepdims=True)
    return (x - m) / jnp.sqrt(v + 1e-5) * g + b


def _transformer_body(wch_ref, x_ref, wq_ref, wk_ref, wv_ref, wo_ref,
                      w1_ref, w2_ref, l1g_ref, l1b_ref, l2g_ref, l2b_ref,
                      zch_ref):
    def mm(a, b):
        return lax.dot_general(
            a.astype(jnp.bfloat16), b.astype(jnp.bfloat16),
            (((1,), (0,)), ((), ())), preferred_element_type=jnp.float32)

    z = x_ref[0]
    impacc = jnp.zeros((1, N), jnp.float32)
    scale = 1.0 / math.sqrt(DH)
    for l in range(L):
        q = mm(z, wq_ref[l])
        k = mm(z, wk_ref[l])
        v = mm(z, wv_ref[l])
        heads = []
        for h in range(H):
            sl = slice(h * DH, (h + 1) * DH)
            qh, kh, vh = q[:, sl], k[:, sl], v[:, sl]
            s = lax.dot_general(
                qh.astype(jnp.bfloat16), kh.astype(jnp.bfloat16),
                (((1,), (1,)), ((), ())),
                preferred_element_type=jnp.float32) * scale
            m = jnp.max(s, axis=-1, keepdims=True)
            e = jnp.exp(s - m)
            den = jnp.sum(e, axis=-1, keepdims=True)
            a = e / den
            impacc = impacc + wch_ref[l * H + h] * jnp.sum(a, axis=0,
                                                           keepdims=True)
            heads.append(mm(a, vh))
        o = mm(jnp.concatenate(heads, axis=1), wo_ref[l])
        z = _ln(z + o, l1g_ref[l], l1b_ref[l])
        f = mm(_leaky(mm(z, w1_ref[l])), w2_ref[l])
        z = _ln(z + f, l2g_ref[l], l2b_ref[l])
    imp = _leaky(impacc * (1.0 / N))
    impc = imp.reshape(N, 1)
    zfw = z * impc
    zcat = jnp.concatenate(
        [z, zfw, impc, jnp.ones((N, 1), jnp.float32),
         jnp.zeros((N, GC - 2 * D - 2), jnp.float32)], axis=1)
    for p in range(NCH):
        zch_ref[p] = zcat[:, p * CC:(p + 1) * CC]


def _run_transformer(x, wq, wk, wv, wo, w1, w2, l1g, l1b, l2g, l2b, w_ch):
    full = lambda a: pl.BlockSpec(a.shape, lambda b: (0,) * a.ndim)
    return pl.pallas_call(
        _transformer_body,
        grid=(B,),
        in_specs=[
            pl.BlockSpec(memory_space=pltpu.SMEM),
            pl.BlockSpec((1, N, D), lambda b: (b, 0, 0)),
            full(wq), full(wk), full(wv), full(wo), full(w1), full(w2),
            full(l1g), full(l1b), full(l2g), full(l2b),
        ],
        out_specs=pl.BlockSpec((NCH, N, CC), lambda b: (0, b, 0)),
        out_shape=jax.ShapeDtypeStruct((NCH, BN, CC), jnp.float32),
    )(w_ch, x, wq, wk, wv, wo, w1, w2, l1g, l1b, l2g, l2b)


_NBUF = 2
_NJ = NB // _NBUF


def _sc_body(*refs):
    tabs = refs[:NCH]
    (gidx_hbm, sidx_hbm, out_hbm,
     gidx_v, sidx_v, rows_v, acc_s, sem, sem2) = refs[NCH:]
    cid = lax.axis_index("c")
    sid = lax.axis_index("s")
    wid = sid * NC + cid

    pltpu.sync_copy(gidx_hbm.at[wid], gidx_v)
    pltpu.sync_copy(sidx_hbm.at[wid], sidx_v)

    for p in range(NCH):
        @pl.loop(0, ZR)
        def _zf(i):
            for j in range(CC // 16):
                rows_v[0, i, pl.ds(j * 16, 16)] = jnp.zeros((16,), jnp.float32)

        for r in range(RPT // ZR):
            pltpu.sync_copy(rows_v.at[0], acc_s.at[pl.ds(sid * RPT + r * ZR, ZR)])
        plsc.subcore_barrier()

        def fire_g(bi, t):
            pltpu.async_copy(tabs[p].at[gidx_v.at[bi]], rows_v.at[t], sem)

        def drain_g(t):
            pltpu.make_async_copy(tabs[p].at[gidx_v.at[0]], rows_v.at[t],
                                  sem).wait()

        def fire_s(bi, t):
            pltpu.async_copy(rows_v.at[t], acc_s.at[sidx_v.at[bi]], sem2,
                             add=True)

        def drain_s(t):
            pltpu.make_async_copy(rows_v.at[t], acc_s.at[sidx_v.at[0]],
                                  sem2).wait()

        for t in range(_NBUF):
            fire_g(t, t)

        @pl.loop(0, _NJ)
        def _eb(jj):
            b0 = _NBUF * jj
            for t in range(_NBUF):
                drain_g(t)
                fire_s(b0 + t, t)
                drain_s(t)

                @pl.when(jj + 1 < _NJ)
                def _():
                    fire_g(b0 + _NBUF + t, t)

        plsc.subcore_barrier()
        pltpu.sync_copy(acc_s.at[pl.ds(sid * RPT, RPT)],
                        out_hbm.at[cid, p, pl.ds(sid * RPT, RPT)])


def _run_sc_phase(tables, gidx, sidx):
    mesh = plsc.VectorSubcoreMesh(core_axis_name="c", subcore_axis_name="s",
                                  num_cores=NC, num_subcores=NS)
    f = pl.kernel(
        _sc_body,
        out_type=jax.ShapeDtypeStruct((NC, NCH, BN, CC), jnp.float32),
        mesh=mesh,
        scratch_types=[
            pltpu.VMEM((NB, KB), jnp.int32),
            pltpu.VMEM((NB, KB), jnp.int32),
            pltpu.VMEM((_NBUF, KB, CC), jnp.float32),
            pltpu.VMEM_SHARED((BN, CC), jnp.float32),
            pltpu.SemaphoreType.DMA,
            pltpu.SemaphoreType.DMA,
        ],
        compiler_params=pltpu.CompilerParams(use_tc_tiling_on_sc=False),
    )
    return f(*tables, gidx.reshape(NW, NB, KB), sidx.reshape(NW, NB, KB))


_RB = 1024


def _phase_b_body(wch_ref, pa_ref, out_ref):
    pa = pa_ref[...]
    s = pa[0] + pa[1]
    cat = jnp.concatenate([s[p] for p in range(NCH)], axis=1)
    s1 = cat[:, :D]
    m2 = cat[:, D:2 * D]
    sw2 = cat[:, 2 * D:2 * D + 1]
    cnt = cat[:, 2 * D + 1:2 * D + 2]
    c = _leaky((wch_ref[0] + wch_ref[1] + wch_ref[2] + wch_ref[3]) / N)
    agg1 = (c * s1) / (c * cnt + 1e-6)
    agg2 = m2 / (sw2 + 1e-6)
    outcat = jnp.concatenate(
        [agg1, agg2, jnp.zeros((_RB, 1), jnp.float32),
         jnp.ones((_RB, 1), jnp.float32),
         jnp.zeros((_RB, GC - 2 * D - 2), jnp.float32)], axis=1)
    for p in range(NCH):
        out_ref[p] = outcat[:, p * CC:(p + 1) * CC]


def _run_phase_b(pa, w_ch):
    nr = BN // _RB
    return pl.pallas_call(
        _phase_b_body,
        grid=(nr,),
        in_specs=[
            pl.BlockSpec(memory_space=pltpu.SMEM),
            pl.BlockSpec((NC, NCH, _RB, CC), lambda r: (0, 0, r, 0)),
        ],
        out_specs=pl.BlockSpec((NCH, _RB, CC), lambda r: (0, r, 0)),
        out_shape=jax.ShapeDtypeStruct((NCH, BN, CC), jnp.float32),
    )(w_ch, pa)


def _phase_d_body(wch_ref, pc_ref, z4_ref, wh1_ref, wh2_ref, out_ref):
    pc = pc_ref[...]
    s = pc[0] + pc[1]
    cat = jnp.concatenate([s[p] for p in range(NCH)], axis=1)
    t1 = cat[:, :D]
    u2 = cat[:, D:2 * D]
    cnt = cat[:, 2 * D + 1:2 * D + 2]
    iloc = 2 * D - (NCH - 1) * CC
    im = z4_ref[...][:, iloc:iloc + 1]
    c = _leaky((wch_ref[0] + wch_ref[1] + wch_ref[2] + wch_ref[3]) / N)
    o1 = (c * t1) / (c * cnt + 1e-6)
    o2 = (im * u2) / (im * cnt + 1e-6)
    out_ref[...] = _leaky(o1 @ wh1_ref[...]) + _leaky(o2 @ wh2_ref[...])


def _run_phase_d(pc, z4, wh1, wh2, w_ch):
    nr = BN // _RB
    full = lambda a: pl.BlockSpec(a.shape, lambda r: (0,) * a.ndim)
    return pl.pallas_call(
        _phase_d_body,
        grid=(nr,),
        in_specs=[
            pl.BlockSpec(memory_space=pltpu.SMEM),
            pl.BlockSpec((NC, NCH, _RB, CC), lambda r: (0, 0, r, 0)),
            pl.BlockSpec((_RB, CC), lambda r: (r, 0)),
            full(wh1), full(wh2),
        ],
        out_specs=pl.BlockSpec((_RB, D), lambda r: (r, 0)),
        out_shape=jax.ShapeDtypeStruct((BN, D), jnp.float32),
    )(w_ch, pc, z4, wh1, wh2)


def kernel(x, edge_index, edge_weight, batch, Wq, Wk, Wv, Wo, W1, W2,
           ln1g, ln1b, ln2g, ln2b, w_ch, Wh1, Wh2):
    del edge_weight, batch
    src = edge_index[0]
    he = edge_index[1]
    zch = _run_transformer(x, Wq, Wk, Wv, Wo, W1, W2,
                           ln1g, ln1b, ln2g, ln2b, w_ch)
    ztabs = [zch[p] for p in range(NCH)]
    pa = _run_sc_phase(ztabs, src, he)
    aggc = _run_phase_b(pa, w_ch)
    atabs = [aggc[p] for p in range(NCH)]
    pc = _run_sc_phase(atabs, he, src)
    return _run_phase_d(pc, zch[NCH - 1], Wh1, Wh2, w_ch)


# speedup vs baseline: 1.1782x; 1.0443x over previous
import functools
import math

import jax
import jax.numpy as jnp
from jax import lax
from jax.experimental import pallas as pl
from jax.experimental.pallas import tpu as pltpu
from jax.experimental.pallas import tpu_sc as plsc

B, N, D = 64, 256, 256
L, H = 2, 2
DH = D // H
FF = 512
E = 262144
BN = B * N

NC, NS = 2, 16
NW = NC * NS
CC = 80
NCH = 7
GC = NCH * CC
EW = E // NW
KB = 128
NB = EW // KB
ZR = 128
RPT = BN // NS


def _leaky(x):
    return jnp.where(x >= 0, x, 0.01 * x)


def _ln(x, g, b):
    m = jnp.mean(x, axis=-1, keepdims=True)
    v = jnp.mean((x - m) ** 2, axis=-1, keepdims=True)
    return (x - m) / jnp.sqrt(v + 1e-5) * g + b


def _transformer_body(wch_ref, x_ref, wq_ref, wk_ref, wv_ref, wo_ref,
                      w1_ref, w2_ref, l1g_ref, l1b_ref, l2g_ref, l2b_ref,
                      *zch_refs):
    def mm(a, b):
        return lax.dot_general(
            a.astype(jnp.bfloat16), b.astype(jnp.bfloat16),
            (((1,), (0,)), ((), ())), preferred_element_type=jnp.float32)

    z = x_ref[0]
    impacc = jnp.zeros((1, N), jnp.float32)
    scale = 1.0 / math.sqrt(DH)
    for l in range(L):
        q = mm(z, wq_ref[l])
        k = mm(z, wk_ref[l])
        v = mm(z, wv_ref[l])
        heads = []
        for h in range(H):
            sl = slice(h * DH, (h + 1) * DH)
            qh, kh, vh = q[:, sl], k[:, sl], v[:, sl]
            s = lax.dot_general(
                qh.astype(jnp.bfloat16), kh.astype(jnp.bfloat16),
                (((1,), (1,)), ((), ())),
                preferred_element_type=jnp.float32) * scale
            m = jnp.max(s, axis=-1, keepdims=True)
            e = jnp.exp(s - m)
            den = jnp.sum(e, axis=-1, keepdims=True)
            a = e / den
            impacc = impacc + wch_ref[l * H + h] * jnp.sum(a, axis=0,
                                                           keepdims=True)
            heads.append(mm(a, vh))
        o = mm(jnp.concatenate(heads, axis=1), wo_ref[l])
        z = _ln(z + o, l1g_ref[l], l1b_ref[l])
        f = mm(_leaky(mm(z, w1_ref[l])), w2_ref[l])
        z = _ln(z + f, l2g_ref[l], l2b_ref[l])
    imp = _leaky(impacc * (1.0 / N))
    impc = imp.reshape(N, 1)
    zfw = z * impc
    zcat = jnp.concatenate(
        [z, zfw, impc, jnp.ones((N, 1), jnp.float32),
         jnp.zeros((N, GC - 2 * D - 2), jnp.float32)], axis=1)
    for p in range(NCH):
        zch_refs[p][...] = zcat[:, p * CC:(p + 1) * CC]


def _run_transformer(x, wq, wk, wv, wo, w1, w2, l1g, l1b, l2g, l2b, w_ch):
    full = lambda a: pl.BlockSpec(a.shape, lambda b: (0,) * a.ndim)
    return pl.pallas_call(
        _transformer_body,
        grid=(B,),
        in_specs=[
            pl.BlockSpec(memory_space=pltpu.SMEM),
            pl.BlockSpec((1, N, D), lambda b: (b, 0, 0)),
            full(wq), full(wk), full(wv), full(wo), full(w1), full(w2),
            full(l1g), full(l1b), full(l2g), full(l2b),
        ],
        out_specs=[pl.BlockSpec((N, CC), lambda b: (b, 0))] * NCH,
        out_shape=[jax.ShapeDtypeStruct((BN, CC), jnp.float32)] * NCH,
    )(w_ch, x, wq, wk, wv, wo, w1, w2, l1g, l1b, l2g, l2b)


_NBUF = 2
_NJ = NB // _NBUF


def _sc_body(*refs):
    tabs = refs[:NCH]
    (gidx_hbm, sidx_hbm, out_hbm,
     gidx_v, sidx_v, rows_v, acc_s, sem, sem2) = refs[NCH:]
    cid = lax.axis_index("c")
    sid = lax.axis_index("s")
    wid = sid * NC + cid

    pltpu.sync_copy(gidx_hbm.at[wid], gidx_v)
    pltpu.sync_copy(sidx_hbm.at[wid], sidx_v)

    for p in range(NCH):
        @pl.loop(0, ZR)
        def _zf(i):
            for j in range(CC // 16):
                rows_v[0, i, pl.ds(j * 16, 16)] = jnp.zeros((16,), jnp.float32)

        for r in range(RPT // ZR):
            pltpu.sync_copy(rows_v.at[0], acc_s.at[pl.ds(sid * RPT + r * ZR, ZR)])
        plsc.subcore_barrier()

        def fire_g(bi, t):
            pltpu.async_copy(tabs[p].at[gidx_v.at[bi]], rows_v.at[t], sem)

        def drain_g(t):
            pltpu.make_async_copy(tabs[p].at[gidx_v.at[0]], rows_v.at[t],
                                  sem).wait()

        def fire_s(bi, t):
            pltpu.async_copy(rows_v.at[t], acc_s.at[sidx_v.at[bi]], sem2,
                             add=True)

        def drain_s(t):
            pltpu.make_async_copy(rows_v.at[t], acc_s.at[sidx_v.at[0]],
                                  sem2).wait()

        for t in range(_NBUF):
            fire_g(t, t)

        @pl.loop(0, _NJ)
        def _eb(jj):
            b0 = _NBUF * jj
            for t in range(_NBUF):
                drain_g(t)
                fire_s(b0 + t, t)
                drain_s(t)

                @pl.when(jj + 1 < _NJ)
                def _():
                    fire_g(b0 + _NBUF + t, t)

        plsc.subcore_barrier()
        pltpu.sync_copy(acc_s.at[pl.ds(sid * RPT, RPT)],
                        out_hbm.at[cid, p, pl.ds(sid * RPT, RPT)])


def _run_sc_phase(tables, gidx, sidx):
    mesh = plsc.VectorSubcoreMesh(core_axis_name="c", subcore_axis_name="s",
                                  num_cores=NC, num_subcores=NS)
    f = pl.kernel(
        _sc_body,
        out_type=jax.ShapeDtypeStruct((NC, NCH, BN, CC), jnp.float32),
        mesh=mesh,
        scratch_types=[
            pltpu.VMEM((NB, KB), jnp.int32),
            pltpu.VMEM((NB, KB), jnp.int32),
            pltpu.VMEM((_NBUF, KB, CC), jnp.float32),
            pltpu.VMEM_SHARED((BN, CC), jnp.float32),
            pltpu.SemaphoreType.DMA,
            pltpu.SemaphoreType.DMA,
        ],
        compiler_params=pltpu.CompilerParams(use_tc_tiling_on_sc=False),
    )
    return f(*tables, gidx.reshape(NW, NB, KB), sidx.reshape(NW, NB, KB))


_RB = 1024


def _phase_b_body(wch_ref, pa_ref, *out_refs):
    pa = pa_ref[...]
    s = pa[0] + pa[1]
    cat = jnp.concatenate([s[p] for p in range(NCH)], axis=1)
    s1 = cat[:, :D]
    m2 = cat[:, D:2 * D]
    sw2 = cat[:, 2 * D:2 * D + 1]
    cnt = cat[:, 2 * D + 1:2 * D + 2]
    c = _leaky((wch_ref[0] + wch_ref[1] + wch_ref[2] + wch_ref[3]) / N)
    agg1 = (c * s1) / (c * cnt + 1e-6)
    agg2 = m2 / (sw2 + 1e-6)
    outcat = jnp.concatenate(
        [agg1, agg2, jnp.zeros((_RB, 1), jnp.float32),
         jnp.ones((_RB, 1), jnp.float32),
         jnp.zeros((_RB, GC - 2 * D - 2), jnp.float32)], axis=1)
    for p in range(NCH):
        out_refs[p][...] = outcat[:, p * CC:(p + 1) * CC]


def _run_phase_b(pa, w_ch):
    nr = BN // _RB
    return pl.pallas_call(
        _phase_b_body,
        grid=(nr,),
        in_specs=[
            pl.BlockSpec(memory_space=pltpu.SMEM),
            pl.BlockSpec((NC, NCH, _RB, CC), lambda r: (0, 0, r, 0)),
        ],
        out_specs=[pl.BlockSpec((_RB, CC), lambda r: (r, 0))] * NCH,
        out_shape=[jax.ShapeDtypeStruct((BN, CC), jnp.float32)] * NCH,
    )(w_ch, pa)


def _phase_d_body(wch_ref, pc_ref, z4_ref, wh1_ref, wh2_ref, out_ref):
    pc = pc_ref[...]
    s = pc[0] + pc[1]
    cat = jnp.concatenate([s[p] for p in range(NCH)], axis=1)
    t1 = cat[:, :D]
    u2 = cat[:, D:2 * D]
    cnt = cat[:, 2 * D + 1:2 * D + 2]
    iloc = 2 * D - (NCH - 1) * CC
    im = z4_ref[...][:, iloc:iloc + 1]
    c = _leaky((wch_ref[0] + wch_ref[1] + wch_ref[2] + wch_ref[3]) / N)
    o1 = (c * t1) / (c * cnt + 1e-6)
    o2 = (im * u2) / (im * cnt + 1e-6)
    out_ref[...] = _leaky(o1 @ wh1_ref[...]) + _leaky(o2 @ wh2_ref[...])


def _run_phase_d(pc, z4, wh1, wh2, w_ch):
    nr = BN // _RB
    full = lambda a: pl.BlockSpec(a.shape, lambda r: (0,) * a.ndim)
    return pl.pallas_call(
        _phase_d_body,
        grid=(nr,),
        in_specs=[
            pl.BlockSpec(memory_space=pltpu.SMEM),
            pl.BlockSpec((NC, NCH, _RB, CC), lambda r: (0, 0, r, 0)),
            pl.BlockSpec((_RB, CC), lambda r: (r, 0)),
            full(wh1), full(wh2),
        ],
        out_specs=pl.BlockSpec((_RB, D), lambda r: (r, 0)),
        out_shape=jax.ShapeDtypeStruct((BN, D), jnp.float32),
    )(w_ch, pc, z4, wh1, wh2)


def kernel(x, edge_index, edge_weight, batch, Wq, Wk, Wv, Wo, W1, W2,
           ln1g, ln1b, ln2g, ln2b, w_ch, Wh1, Wh2):
    del edge_weight, batch
    src = edge_index[0]
    he = edge_index[1]
    ztabs = _run_transformer(x, Wq, Wk, Wv, Wo, W1, W2,
                             ln1g, ln1b, ln2g, ln2b, w_ch)
    pa = _run_sc_phase(ztabs, src, he)
    atabs = _run_phase_b(pa, w_ch)
    pc = _run_sc_phase(atabs, he, src)
    return _run_phase_d(pc, ztabs[NCH - 1], Wh1, Wh2, w_ch)
